# baseline (speedup 1.0000x reference)
import functools

import jax
import jax.numpy as jnp
from jax import lax
from jax.experimental import pallas as pl
from jax.experimental.pallas import tpu as pltpu
from jax.experimental.pallas import tpu_sc as plsc

NV = 1_000_000
FEATURES = 64
B_TOK = 4096 * 50
LANE = 128
NC, NS = 2, 16
NW = NC * NS
B_PER_W = B_TOK // NW
NJ = 5
CHUNK = NJ * LANE
NCHUNK = B_PER_W // CHUNK

TW = 512
NWIN = NV // TW
REM = NV - NWIN * TW


def _transpose_body(tview_hbm, tail_hbm, tlin_hbm, in_v, out_v, sem):
    wid = lax.axis_index("s") * NC + lax.axis_index("c")
    lane16 = lax.broadcasted_iota(jnp.int32, (16,), 0)

    def do_window(c0, width):
        for g in range(8):
            pltpu.sync_copy(
                tview_hbm.at[pl.ds(8 * g, 8), pl.ds(c0, width)],
                in_v.at[pl.ds(8 * g, 8), pl.ds(0, width)])

        def per_c(c, _):
            cols = jnp.full((16,), 0, jnp.int32) + c
            for k in range(4):
                rows = lane16 + (16 * k)
                x = plsc.load_gather(in_v, [rows, cols])
                out_v[pl.ds(c * FEATURES + 16 * k, 16)] = x
            return _
        lax.fori_loop(0, width, per_c, 0, unroll=4)
        pltpu.sync_copy(out_v.at[pl.ds(0, width * FEATURES)],
                        tlin_hbm.at[pl.ds(c0 * FEATURES, width * FEATURES)])

    lo = (NWIN * wid) // NW
    hi = (NWIN * (wid + 1)) // NW

    def win(w, _):
        do_window(w * TW, TW)
        return _
    lax.fori_loop(lo, hi, win, 0)

    @pl.when(wid == NW - 1)
    def _():
        pltpu.sync_copy(tail_hbm, out_v.at[pl.ds(0, REM * FEATURES)])
        pltpu.sync_copy(out_v.at[pl.ds(0, REM * FEATURES)],
                        tlin_hbm.at[pl.ds(NWIN * TW * FEATURES,
                                          REM * FEATURES)])


def _gather_body(idx_hbm, table_hbm, out_hbm, idx_v, rows0, rows1,
                 sg0, sg1, so0, so1):
    wid = lax.axis_index("s") * NC + lax.axis_index("c")
    base = wid * B_PER_W
    pltpu.sync_copy(idx_hbm.at[pl.ds(base, B_PER_W)], idx_v)

    rows = (rows0, rows1)
    sg = (sg0, sg1)
    so = (so0, so1)
    gather_h = {}
    write_h = {}

    def fire_gathers(c):
        b = c % 2
        hs = []
        for j in range(NJ):
            k = c * NJ + j
            hs.append(pltpu.async_copy(
                table_hbm.at[idx_v.at[pl.ds(k * LANE, LANE)]],
                rows[b].at[pl.ds(j * LANE, LANE)],
                sg[b]))
        gather_h[c] = hs

    fire_gathers(0)
    for c in range(NCHUNK):
        b = c % 2
        if c + 1 < NCHUNK:
            if c >= 1:
                write_h[c - 1].wait()
            fire_gathers(c + 1)
        for h in gather_h[c]:
            h.wait()
        write_h[c] = pltpu.async_copy(
            rows[b], out_hbm.at[pl.ds(base + c * CHUNK, CHUNK)], so[b])
    write_h[NCHUNK - 2].wait()
    write_h[NCHUNK - 1].wait()


@jax.jit
def _embed_lookup(idx1d, tview, tail):
    mesh = plsc.VectorSubcoreMesh(core_axis_name="c", subcore_axis_name="s")
    transpose_fn = functools.partial(
        pl.kernel,
        out_type=jax.ShapeDtypeStruct((NV * FEATURES,), jnp.float32),
        mesh=mesh,
        scratch_types=[
            pltpu.VMEM((FEATURES, TW), jnp.float32),
            pltpu.VMEM((TW * FEATURES,), jnp.float32),
            pltpu.SemaphoreType.DMA,
        ],
        compiler_params=pltpu.CompilerParams(needs_layout_passes=False),
    )(_transpose_body)
    tlin = transpose_fn(tview, tail)
    table = tlin.reshape(NV, FEATURES)

    gather_fn = functools.partial(
        pl.kernel,
        out_type=jax.ShapeDtypeStruct((B_TOK, FEATURES), jnp.float32),
        mesh=mesh,
        scratch_types=[
            pltpu.VMEM((B_PER_W,), jnp.int32),
            pltpu.VMEM((CHUNK, FEATURES), jnp.float32),
            pltpu.VMEM((CHUNK, FEATURES), jnp.float32),
            pltpu.SemaphoreType.DMA,
            pltpu.SemaphoreType.DMA,
            pltpu.SemaphoreType.DMA,
            pltpu.SemaphoreType.DMA,
        ],
        compiler_params=pltpu.CompilerParams(use_tc_tiling_on_sc=False),
    )(_gather_body)
    return gather_fn(idx1d, table)


def kernel(inputs, embedding):
    idx1d = inputs.reshape(B_TOK).astype(jnp.int32)
    tail = embedding[NWIN * TW:, :].reshape(REM * FEATURES)
    out = _embed_lookup(idx1d, embedding.T, tail)
    return out.reshape(inputs.shape + (embedding.shape[-1],))


# speedup vs baseline: 1.3281x; 1.3281x over previous
import functools

import jax
import jax.numpy as jnp
from jax import lax
from jax.experimental import pallas as pl
from jax.experimental.pallas import tpu as pltpu
from jax.experimental.pallas import tpu_sc as plsc

NV = 1_000_000
FEATURES = 64
B_TOK = 4096 * 50
LANE = 128
NC, NS = 2, 16
NW = NC * NS
B_PER_W = B_TOK // NW
NJ = 5
CHUNK = NJ * LANE
NCHUNK = B_PER_W // CHUNK

TW = 512
NWIN = NV // TW
REM = NV - NWIN * TW


def _transpose_body(tview_hbm, tail_hbm, tlin_hbm, in_v, out_v, sem):
    wid = lax.axis_index("s") * NC + lax.axis_index("c")
    lane16 = lax.broadcasted_iota(jnp.int32, (16,), 0)

    def do_window(c0, width):
        hs = []
        for g in range(8):
            hs.append(pltpu.async_copy(
                tview_hbm.at[pl.ds(8 * g, 8), pl.ds(c0, width)],
                in_v.at[pl.ds(8 * g, 8), pl.ds(0, width)], sem))
        for h in hs:
            h.wait()

        def per_f(f, _):
            def per_k(k, idxv):
                x = in_v[f, pl.ds(16 * k, 16)]
                plsc.store_scatter(out_v, [idxv], x)
                return idxv + (16 * FEATURES)
            lax.fori_loop(0, width // 16, per_k, lane16 * FEATURES + f,
                          unroll=8)
            return _
        lax.fori_loop(0, FEATURES, per_f, 0)
        pltpu.sync_copy(out_v.at[pl.ds(0, width * FEATURES)],
                        tlin_hbm.at[pl.ds(c0 * FEATURES, width * FEATURES)])

    lo = (NWIN * wid) // NW
    hi = (NWIN * (wid + 1)) // NW

    def win(w, _):
        do_window(w * TW, TW)
        return _
    lax.fori_loop(lo, hi, win, 0)

    @pl.when(wid == NW - 1)
    def _():
        pltpu.sync_copy(tail_hbm, out_v.at[pl.ds(0, REM * FEATURES)])
        pltpu.sync_copy(out_v.at[pl.ds(0, REM * FEATURES)],
                        tlin_hbm.at[pl.ds(NWIN * TW * FEATURES,
                                          REM * FEATURES)])


def _gather_body(idx_hbm, table_hbm, out_hbm, idx_v, rows0, rows1,
                 sg0, sg1, so0, so1):
    wid = lax.axis_index("s") * NC + lax.axis_index("c")
    base = wid * B_PER_W
    pltpu.sync_copy(idx_hbm.at[pl.ds(base, B_PER_W)], idx_v)

    rows = (rows0, rows1)
    sg = (sg0, sg1)
    so = (so0, so1)
    gather_h = {}
    write_h = {}

    def fire_gathers(c):
        b = c % 2
        hs = []
        for j in range(NJ):
            k = c * NJ + j
            hs.append(pltpu.async_copy(
                table_hbm.at[idx_v.at[pl.ds(k * LANE, LANE)]],
                rows[b].at[pl.ds(j * LANE, LANE)],
                sg[b]))
        gather_h[c] = hs

    fire_gathers(0)
    for c in range(NCHUNK):
        b = c % 2
        if c + 1 < NCHUNK:
            if c >= 1:
                write_h[c - 1].wait()
            fire_gathers(c + 1)
        for h in gather_h[c]:
            h.wait()
        write_h[c] = pltpu.async_copy(
            rows[b], out_hbm.at[pl.ds(base + c * CHUNK, CHUNK)], so[b])
    write_h[NCHUNK - 2].wait()
    write_h[NCHUNK - 1].wait()


@jax.jit
def _embed_lookup(idx1d, tview, tail):
    mesh = plsc.VectorSubcoreMesh(core_axis_name="c", subcore_axis_name="s")
    transpose_fn = functools.partial(
        pl.kernel,
        out_type=jax.ShapeDtypeStruct((NV * FEATURES,), jnp.float32),
        mesh=mesh,
        scratch_types=[
            pltpu.VMEM((FEATURES, TW), jnp.float32),
            pltpu.VMEM((TW * FEATURES,), jnp.float32),
            pltpu.SemaphoreType.DMA,
        ],
        compiler_params=pltpu.CompilerParams(needs_layout_passes=False),
    )(_transpose_body)
    tlin = transpose_fn(tview, tail)
    table = tlin.reshape(NV, FEATURES)

    gather_fn = functools.partial(
        pl.kernel,
        out_type=jax.ShapeDtypeStruct((B_TOK, FEATURES), jnp.float32),
        mesh=mesh,
        scratch_types=[
            pltpu.VMEM((B_PER_W,), jnp.int32),
            pltpu.VMEM((CHUNK, FEATURES), jnp.float32),
            pltpu.VMEM((CHUNK, FEATURES), jnp.float32),
            pltpu.SemaphoreType.DMA,
            pltpu.SemaphoreType.DMA,
            pltpu.SemaphoreType.DMA,
            pltpu.SemaphoreType.DMA,
        ],
        compiler_params=pltpu.CompilerParams(use_tc_tiling_on_sc=False),
    )(_gather_body)
    return gather_fn(idx1d, table)


def kernel(inputs, embedding):
    idx1d = inputs.reshape(B_TOK).astype(jnp.int32)
    tail = embedding[NWIN * TW:, :].reshape(REM * FEATURES)
    out = _embed_lookup(idx1d, embedding.T, tail)
    return out.reshape(inputs.shape + (embedding.shape[-1],))


# speedup vs baseline: 1.5928x; 1.1993x over previous
import functools

import jax
import jax.numpy as jnp
from jax import lax
from jax.experimental import pallas as pl
from jax.experimental.pallas import tpu as pltpu
from jax.experimental.pallas import tpu_sc as plsc

NV = 1_000_000
FEATURES = 64
B_TOK = 4096 * 50
LANE = 128
NC, NS = 2, 16
NW = NC * NS
B_PER_W = B_TOK // NW
NJ = 5
CHUNK = NJ * LANE
NCHUNK = B_PER_W // CHUNK

TW = 512
NWIN = NV // TW
REM = NV - NWIN * TW


def _transpose_body(tview_hbm, tail_hbm, tlin_hbm, in_v, out_v, sem, semt):
    wid = lax.axis_index("s") * NC + lax.axis_index("c")
    lane16 = lax.broadcasted_iota(jnp.int32, (16,), 0)

    def do_window(c0, width):
        hs = []
        for g in range(8):
            hs.append(pltpu.async_copy(
                tview_hbm.at[pl.ds(8 * g, 8), pl.ds(c0, width)],
                in_v.at[pl.ds(8 * g, 8), pl.ds(0, width)], sem))
        for h in hs:
            h.wait()

        def per_c(cb, _):
            rows2 = 16 * cb + lane16

            def per_d(d, _2):
                fl0 = jnp.bitwise_and(lane16 + d, 15)
                for fb in range(FEATURES // 16):
                    fl = fl0 + 16 * fb
                    x = plsc.load_gather(in_v, [fl, rows2])
                    plsc.store_scatter(out_v, [rows2, fl], x)
                return _2
            lax.fori_loop(0, 16, per_d, 0, unroll=4)
            return _
        lax.fori_loop(0, width // 16, per_c, 0)
        pltpu.sync_copy(out_v.at[pl.ds(0, width), :],
                        tlin_hbm.at[pl.ds(c0, width), :])

    lo = (NWIN * wid) // NW
    hi = (NWIN * (wid + 1)) // NW

    def win(w, _):
        do_window(w * TW, TW)
        return _
    lax.fori_loop(lo, hi, win, 0)

    @pl.when(wid == NW - 1)
    def _():
        pltpu.sync_copy(tail_hbm, out_v.at[pl.ds(0, REM), :])
        pltpu.sync_copy(out_v.at[pl.ds(0, REM), :],
                        tlin_hbm.at[pl.ds(NWIN * TW, REM), :])


def _gather_body(idx_hbm, table_hbm, out_hbm, idx_v, rows0, rows1,
                 sg0, sg1, so0, so1):
    wid = lax.axis_index("s") * NC + lax.axis_index("c")
    base = wid * B_PER_W
    pltpu.sync_copy(idx_hbm.at[pl.ds(base, B_PER_W)], idx_v)

    rows = (rows0, rows1)
    sg = (sg0, sg1)
    so = (so0, so1)
    gather_h = {}
    write_h = {}

    def fire_gathers(c):
        b = c % 2
        hs = []
        for j in range(NJ):
            k = c * NJ + j
            hs.append(pltpu.async_copy(
                table_hbm.at[idx_v.at[pl.ds(k * LANE, LANE)]],
                rows[b].at[pl.ds(j * LANE, LANE)],
                sg[b]))
        gather_h[c] = hs

    fire_gathers(0)
    for c in range(NCHUNK):
        b = c % 2
        if c + 1 < NCHUNK:
            if c >= 1:
                write_h[c - 1].wait()
            fire_gathers(c + 1)
        for h in gather_h[c]:
            h.wait()
        write_h[c] = pltpu.async_copy(
            rows[b], out_hbm.at[pl.ds(base + c * CHUNK, CHUNK)], so[b])
    write_h[NCHUNK - 2].wait()
    write_h[NCHUNK - 1].wait()


@jax.jit
def _embed_lookup(idx1d, tview, tail):
    mesh = plsc.VectorSubcoreMesh(core_axis_name="c", subcore_axis_name="s")
    transpose_fn = functools.partial(
        pl.kernel,
        out_type=jax.ShapeDtypeStruct((NV, FEATURES), jnp.float32),
        mesh=mesh,
        scratch_types=[
            pltpu.VMEM((FEATURES, TW), jnp.float32),
            pltpu.VMEM((TW, FEATURES), jnp.float32),
            pltpu.SemaphoreType.DMA,
            pltpu.SemaphoreType.DMA,
        ],
        compiler_params=pltpu.CompilerParams(needs_layout_passes=False),
    )(_transpose_body)
    table = transpose_fn(tview, tail)

    gather_fn = functools.partial(
        pl.kernel,
        out_type=jax.ShapeDtypeStruct((B_TOK, FEATURES), jnp.float32),
        mesh=mesh,
        scratch_types=[
            pltpu.VMEM((B_PER_W,), jnp.int32),
            pltpu.VMEM((CHUNK, FEATURES), jnp.float32),
            pltpu.VMEM((CHUNK, FEATURES), jnp.float32),
            pltpu.SemaphoreType.DMA,
            pltpu.SemaphoreType.DMA,
            pltpu.SemaphoreType.DMA,
            pltpu.SemaphoreType.DMA,
        ],
        compiler_params=pltpu.CompilerParams(use_tc_tiling_on_sc=False),
    )(_gather_body)
    return gather_fn(idx1d, table)


def kernel(inputs, embedding):
    idx1d = inputs.reshape(B_TOK).astype(jnp.int32)
    tail = embedding[NWIN * TW:, :]
    out = _embed_lookup(idx1d, embedding.T, tail)
    return out.reshape(inputs.shape + (embedding.shape[-1],))


# speedup vs baseline: 2.0300x; 1.2745x over previous
import functools

import jax
import jax.numpy as jnp
from jax import lax
from jax.experimental import pallas as pl
from jax.experimental.pallas import tpu as pltpu
from jax.experimental.pallas import tpu_sc as plsc

NV = 1_000_000
FEATURES = 64
B_TOK = 4096 * 50
LANE = 128
NC, NS = 2, 16
NW = NC * NS
B_PER_W = B_TOK // NW
NJ = 5
CHUNK = NJ * LANE
NCHUNK = B_PER_W // CHUNK

TW = 256
NWIN = NV // TW
REM = NV - NWIN * TW


def _transpose_body(tview_hbm, tail_hbm, tlin_hbm, in0, in1, o0, o1,
                    si0, si1, so0, so1):
    wid = lax.axis_index("s") * NC + lax.axis_index("c")
    lane16 = lax.broadcasted_iota(jnp.int32, (16,), 0)
    ins = (in0, in1)
    outs = (o0, o1)
    sis = (si0, si1)
    sos = (so0, so1)

    def fire_stage(w, b):
        for g in range(8):
            pltpu.async_copy(
                tview_hbm.at[pl.ds(8 * g, 8), pl.ds(w * TW, TW)],
                ins[b].at[pl.ds(8 * g, 8), :], sis[b])

    def wait_stage(b):
        for g in range(8):
            pltpu.make_async_copy(
                tview_hbm.at[pl.ds(0, 8), pl.ds(0, TW)],
                ins[b].at[pl.ds(0, 8), :], sis[b]).wait()

    def fire_out(w, b):
        pltpu.async_copy(outs[b], tlin_hbm.at[pl.ds(w * TW, TW), :], sos[b])

    def wait_out(b):
        pltpu.make_async_copy(
            outs[b], tlin_hbm.at[pl.ds(0, TW), :], sos[b]).wait()

    def compute(b):
        def per_c(cb, _):
            rows2 = 16 * cb + lane16

            def per_d(d, _2):
                fl0 = jnp.bitwise_and(lane16 + d, 15)
                for fb in range(FEATURES // 16):
                    fl = fl0 + 16 * fb
                    x = plsc.load_gather(ins[b], [fl, rows2])
                    plsc.store_scatter(outs[b], [rows2, fl], x)
                return _2
            lax.fori_loop(0, 16, per_d, 0, unroll=4)
            return _
        lax.fori_loop(0, TW // 16, per_c, 0)

    lo = (NWIN * wid) // NW
    hi = (NWIN * (wid + 1)) // NW
    n = hi - lo

    fire_stage(lo, 0)
    fire_stage(lo + 1, 1)

    def pair(i, acc):
        for b in range(2):
            w = lo + 2 * i + b

            @pl.when(w < hi)
            def _step():
                wait_stage(b)

                @pl.when(w - 2 >= lo)
                def _wo():
                    wait_out(b)
                compute(b)
                fire_out(w, b)

                @pl.when(w + 2 < hi)
                def _fs():
                    fire_stage(w + 2, b)
        return acc
    lax.fori_loop(0, (n + 1) // 2, pair, 0)
    wait_out(0)
    wait_out(1)

    @pl.when(wid == NW - 1)
    def _():
        pltpu.sync_copy(tail_hbm, o0.at[pl.ds(0, REM), :])
        pltpu.sync_copy(o0.at[pl.ds(0, REM), :],
                        tlin_hbm.at[pl.ds(NWIN * TW, REM), :])


def _gather_body(idx_hbm, table_hbm, out_hbm, idx_v, rows0, rows1,
                 sg0, sg1, so0, so1):
    wid = lax.axis_index("s") * NC + lax.axis_index("c")
    base = wid * B_PER_W
    pltpu.sync_copy(idx_hbm.at[pl.ds(base, B_PER_W)], idx_v)

    rows = (rows0, rows1)
    sg = (sg0, sg1)
    so = (so0, so1)
    gather_h = {}
    write_h = {}

    def fire_gathers(c):
        b = c % 2
        hs = []
        for j in range(NJ):
            k = c * NJ + j
            hs.append(pltpu.async_copy(
                table_hbm.at[idx_v.at[pl.ds(k * LANE, LANE)]],
                rows[b].at[pl.ds(j * LANE, LANE)],
                sg[b]))
        gather_h[c] = hs

    fire_gathers(0)
    for c in range(NCHUNK):
        b = c % 2
        if c + 1 < NCHUNK:
            if c >= 1:
                write_h[c - 1].wait()
            fire_gathers(c + 1)
        for h in gather_h[c]:
            h.wait()
        write_h[c] = pltpu.async_copy(
            rows[b], out_hbm.at[pl.ds(base + c * CHUNK, CHUNK)], so[b])
    write_h[NCHUNK - 2].wait()
    write_h[NCHUNK - 1].wait()


@jax.jit
def _embed_lookup(idx1d, tview, tail):
    mesh = plsc.VectorSubcoreMesh(core_axis_name="c", subcore_axis_name="s")
    transpose_fn = functools.partial(
        pl.kernel,
        out_type=jax.ShapeDtypeStruct((NV, FEATURES), jnp.float32),
        mesh=mesh,
        scratch_types=[
            pltpu.VMEM((FEATURES, TW), jnp.float32),
            pltpu.VMEM((FEATURES, TW), jnp.float32),
            pltpu.VMEM((TW, FEATURES), jnp.float32),
            pltpu.VMEM((TW, FEATURES), jnp.float32),
            pltpu.SemaphoreType.DMA,
            pltpu.SemaphoreType.DMA,
            pltpu.SemaphoreType.DMA,
            pltpu.SemaphoreType.DMA,
        ],
        compiler_params=pltpu.CompilerParams(needs_layout_passes=False),
    )(_transpose_body)
    table = transpose_fn(tview, tail)

    gather_fn = functools.partial(
        pl.kernel,
        out_type=jax.ShapeDtypeStruct((B_TOK, FEATURES), jnp.float32),
        mesh=mesh,
        scratch_types=[
            pltpu.VMEM((B_PER_W,), jnp.int32),
            pltpu.VMEM((CHUNK, FEATURES), jnp.float32),
            pltpu.VMEM((CHUNK, FEATURES), jnp.float32),
            pltpu.SemaphoreType.DMA,
            pltpu.SemaphoreType.DMA,
            pltpu.SemaphoreType.DMA,
            pltpu.SemaphoreType.DMA,
        ],
        compiler_params=pltpu.CompilerParams(use_tc_tiling_on_sc=False),
    )(_gather_body)
    return gather_fn(idx1d, table)


def kernel(inputs, embedding):
    idx1d = inputs.reshape(B_TOK).astype(jnp.int32)
    tail = embedding[NWIN * TW:, :]
    out = _embed_lookup(idx1d, embedding.T, tail)
    return out.reshape(inputs.shape + (embedding.shape[-1],))


# speedup vs baseline: 2.7712x; 1.3651x over previous
import functools

import jax
import jax.numpy as jnp
from jax import lax
from jax.experimental import pallas as pl
from jax.experimental.pallas import tpu as pltpu
from jax.experimental.pallas import tpu_sc as plsc

FEATURES = 64
B_TOK = 4096 * 50
LANE = 128
NC, NS = 2, 16
NW = NC * NS
B_PER_W = B_TOK // NW
NJ = 5
CHUNK = NJ * LANE
NCHUNK = B_PER_W // CHUNK


def _gather_body(idx_hbm, table_hbm, out_hbm, idx_v, rows0, rows1,
                 sg0, sg1, so0, so1):
    wid = lax.axis_index("s") * NC + lax.axis_index("c")
    base = wid * B_PER_W
    pltpu.sync_copy(idx_hbm.at[pl.ds(base, B_PER_W)], idx_v)

    rows = (rows0, rows1)
    sg = (sg0, sg1)
    so = (so0, so1)
    gather_h = {}
    write_h = {}

    def fire_gathers(c):
        b = c % 2
        hs = []
        for j in range(NJ):
            k = c * NJ + j
            hs.append(pltpu.async_copy(
                table_hbm.at[idx_v.at[pl.ds(k * LANE, LANE)]],
                rows[b].at[pl.ds(j * LANE, LANE)],
                sg[b]))
        gather_h[c] = hs

    fire_gathers(0)
    for c in range(NCHUNK):
        b = c % 2
        if c + 1 < NCHUNK:
            if c >= 1:
                write_h[c - 1].wait()
            fire_gathers(c + 1)
        for h in gather_h[c]:
            h.wait()
        write_h[c] = pltpu.async_copy(
            rows[b], out_hbm.at[pl.ds(base + c * CHUNK, CHUNK)], so[b])
    write_h[NCHUNK - 2].wait()
    write_h[NCHUNK - 1].wait()


@jax.jit
def _gather(idx1d, table):
    mesh = plsc.VectorSubcoreMesh(core_axis_name="c", subcore_axis_name="s")
    fn = functools.partial(
        pl.kernel,
        out_type=jax.ShapeDtypeStruct((B_TOK, FEATURES), jnp.float32),
        mesh=mesh,
        scratch_types=[
            pltpu.VMEM((B_PER_W,), jnp.int32),
            pltpu.VMEM((CHUNK, FEATURES), jnp.float32),
            pltpu.VMEM((CHUNK, FEATURES), jnp.float32),
            pltpu.SemaphoreType.DMA,
            pltpu.SemaphoreType.DMA,
            pltpu.SemaphoreType.DMA,
            pltpu.SemaphoreType.DMA,
        ],
        compiler_params=pltpu.CompilerParams(use_tc_tiling_on_sc=False),
    )(_gather_body)
    return fn(idx1d, table)


def kernel(inputs, embedding):
    idx1d = inputs.reshape(B_TOK).astype(jnp.int32)
    out = _gather(idx1d, embedding)
    return out.reshape(inputs.shape + (embedding.shape[-1],))


# speedup vs baseline: 2.8100x; 1.0140x over previous
import functools

import jax
import jax.numpy as jnp
from jax import lax
from jax.experimental import pallas as pl
from jax.experimental.pallas import tpu as pltpu
from jax.experimental.pallas import tpu_sc as plsc

FEATURES = 64
B_TOK = 4096 * 50
LANE = 128
NC, NS = 2, 16
NW = NC * NS
B_PER_W = B_TOK // NW
NJ = 5
CHUNK = NJ * LANE
NCHUNK = B_PER_W // CHUNK


def _gather_body(idx_hbm, table_hbm, out_hbm, idx_v, rows0, rows1,
                 sg0, sg1, so0, so1):
    wid = lax.axis_index("s") * NC + lax.axis_index("c")
    base = wid * B_PER_W
    pltpu.sync_copy(idx_hbm.at[pl.ds(base, B_PER_W)], idx_v)

    rows = (rows0, rows1)
    sg = (sg0, sg1)
    so = (so0, so1)
    gather_h = {}
    write_h = {}

    def fire_gathers(c):
        b = c % 2
        hs = []
        for j in range(NJ):
            k = c * NJ + j
            hs.append(pltpu.async_copy(
                table_hbm.at[idx_v.at[pl.ds(k * LANE, LANE)]],
                rows[b].at[pl.ds(j * LANE, LANE)],
                sg[b]))
        gather_h[c] = hs

    fire_gathers(0)
    for c in range(NCHUNK):
        b = c % 2
        if c + 1 < NCHUNK:
            if c >= 1:
                write_h[c - 1].wait()
            fire_gathers(c + 1)
        for h in gather_h[c]:
            h.wait()
        write_h[c] = pltpu.async_copy(
            rows[b], out_hbm.at[pl.ds(base + c * CHUNK, CHUNK)], so[b])
    write_h[NCHUNK - 2].wait()
    write_h[NCHUNK - 1].wait()


@jax.jit
def _gather(idx1d, table):
    mesh = plsc.VectorSubcoreMesh(core_axis_name="c", subcore_axis_name="s")
    fn = functools.partial(
        pl.kernel,
        out_type=jax.ShapeDtypeStruct((B_TOK, FEATURES), jnp.float32),
        mesh=mesh,
        scratch_types=[
            pltpu.VMEM((B_PER_W,), jnp.int32),
            pltpu.VMEM((CHUNK, FEATURES), jnp.float32),
            pltpu.VMEM((CHUNK, FEATURES), jnp.float32),
            pltpu.SemaphoreType.DMA,
            pltpu.SemaphoreType.DMA,
            pltpu.SemaphoreType.DMA,
            pltpu.SemaphoreType.DMA,
        ],
        compiler_params=pltpu.CompilerParams(use_tc_tiling_on_sc=False),
    )(_gather_body)
    return fn(idx1d, table)


def kernel(inputs, embedding):
    idx1d = inputs.T.reshape(B_TOK).astype(jnp.int32)
    out = _gather(idx1d, embedding)
    return out.reshape(inputs.shape[1], inputs.shape[0],
                       embedding.shape[-1]).transpose(1, 0, 2)
